# baseline (speedup 1.0000x reference)
import functools

import jax
import jax.numpy as jnp
from jax import lax
from jax.experimental import pallas as pl
from jax.experimental.pallas import tpu as pltpu
from jax.experimental.pallas import tpu_sc as plsc

N_NODES = 10000
N_EDGES = 320000
N_QUERIES = 200000
HID = 128
REL = 51

NC = 2
NS = 16
NW = NC * NS

NPAD = 10240
RPT = NPAD // NS

EPW = N_EDGES // NW
ECHUNK = 40
ECHUNKS = EPW // ECHUNK

QPAD = 200704
QPW = QPAD // NW
QCHUNK = 64
QCHUNKS = QPW // QCHUNK
NAUX = 9

_mesh = functools.partial(
    plsc.VectorSubcoreMesh, core_axis_name="c", subcore_axis_name="s",
    num_cores=NC, num_subcores=NS)


def _f32(*shape):
    return jax.ShapeDtypeStruct(shape, jnp.float32)


EBLK = 2000
EBCH = EBLK // ECHUNK


def _sc_edge_body(with_deg, *refs):
    if with_deg:
        (src, dst, a_tab, b_tab, parts, deg16,
         isb, idb, ar0, br0, pb0, ids0, ar1, br1, pb1, ids1, ones, zc,
         acc_sh, deg_sh,
         sga0, sgb0, ssc0, sga1, sgb1, ssc1, sd0, sd1) = refs
    else:
        (src, dst, a_tab, b_tab, parts,
         isb, idb, ar0, br0, pb0, ids0, ar1, br1, pb1, ids1,
         acc_sh,
         sga0, sgb0, ssc0, sga1, sgb1, ssc1) = refs
        sd0 = sd1 = None
    cid = lax.axis_index("c")
    sid = lax.axis_index("s")
    wid = cid * NS + sid
    row0 = sid * RPT
    sets = ((ar0, br0, pb0, ids0, sga0, sgb0, ssc0, sd0),
            (ar1, br1, pb1, ids1, sga1, sgb1, ssc1, sd1))

    zv = jnp.zeros((16,), jnp.float32)
    zi = jnp.zeros((16,), jnp.int32)
    for i in range(ECHUNK):
        for j in range(HID // 16):
            pb0[i, pl.ds(j * 16, 16)] = zv
            pb1[i, pl.ds(j * 16, 16)] = zv
    for j in range(3):
        off = (0, 16, 24)[j]
        ids0[pl.ds(off, 16)] = zi
        ids1[pl.ds(off, 16)] = zi
    for k in range(RPT // ECHUNK):
        pltpu.sync_copy(pb0, acc_sh.at[pl.ds(row0 + k * ECHUNK, ECHUNK)])
    pltpu.async_copy(pb0, acc_sh.at[ids0], ssc0, add=True)
    pltpu.async_copy(pb1, acc_sh.at[ids1], ssc1, add=True)
    if with_deg:
        for i in range(ECHUNK):
            zc[i, :] = zv
        for k in range(RPT // ECHUNK):
            pltpu.sync_copy(zc, deg_sh.at[pl.ds(row0 + k * ECHUNK, ECHUNK)])
        pltpu.async_copy(zc, deg_sh.at[ids0], sd0, add=True)
        pltpu.async_copy(zc, deg_sh.at[ids1], sd1, add=True)
        for i in range(ECHUNK):
            ones[i, :] = jnp.full((16,), 1.0, jnp.float32)
    plsc.subcore_barrier()

    def start_g(S, t):
        off = t * ECHUNK
        pltpu.async_copy(a_tab.at[isb.at[pl.ds(off, ECHUNK)]], S[0], S[4])
        pltpu.async_copy(b_tab.at[idb.at[pl.ds(off, ECHUNK)]], S[1], S[5])

    def wait_g(S):
        pltpu.make_async_copy(a_tab.at[isb.at[pl.ds(0, ECHUNK)]],
                              S[0], S[4]).wait()
        pltpu.make_async_copy(b_tab.at[idb.at[pl.ds(0, ECHUNK)]],
                              S[1], S[5]).wait()

    def wait_s(S):
        pltpu.make_async_copy(S[2], acc_sh.at[S[3]], S[6]).wait()
        if with_deg:
            pltpu.make_async_copy(ones, deg_sh.at[S[3]], S[7]).wait()

    def finish(S, t):
        ar, br, pb, ids = S[0], S[1], S[2], S[3]
        off = t * ECHUNK
        ids[pl.ds(0, 16)] = idb[pl.ds(off, 16)]
        ids[pl.ds(16, 16)] = idb[pl.ds(off + 16, 16)]
        ids[pl.ds(24, 16)] = idb[pl.ds(off + 24, 16)]

        def row(i, c):
            for j in range(HID // 16):
                sl = pl.ds(j * 16, 16)
                pb[i, sl] = jnp.maximum(ar[i, sl] + br[i, sl], 0.0)
            return c

        lax.fori_loop(0, ECHUNK, row, 0)
        pltpu.async_copy(pb, acc_sh.at[ids], S[6], add=True)
        if with_deg:
            pltpu.async_copy(ones, deg_sh.at[ids], S[7], add=True)

    def stage(st, carry):
        base = wid * EPW + st * EBLK
        pltpu.sync_copy(src.at[pl.ds(base, EBLK)], isb)
        pltpu.sync_copy(dst.at[pl.ds(base, EBLK)], idb)
        start_g(sets[0], 0)

        def body(i, c):
            t0 = 2 * i
            A, B = sets
            start_g(B, t0 + 1)
            wait_g(A)
            wait_s(A)
            finish(A, t0)

            @pl.when(i < (EBCH // 2 - 1))
            def _():
                start_g(A, t0 + 2)

            wait_g(B)
            wait_s(B)
            finish(B, t0 + 1)
            return c

        lax.fori_loop(0, EBCH // 2, body, 0)
        return carry

    lax.fori_loop(0, EPW // EBLK, stage, 0)
    wait_s(sets[0])
    wait_s(sets[1])
    plsc.subcore_barrier()
    pltpu.sync_copy(acc_sh.at[pl.ds(row0, RPT)],
                    parts.at[cid, pl.ds(row0, RPT)])
    if with_deg:
        pltpu.sync_copy(deg_sh.at[pl.ds(row0, RPT)],
                        deg16.at[cid, pl.ds(row0, RPT)])


def _make_sc_edge(with_deg):
    outs = [_f32(NC, NPAD, HID)]
    scratch = [
        pltpu.VMEM((EBLK,), jnp.int32),
        pltpu.VMEM((EBLK,), jnp.int32),
        pltpu.VMEM((ECHUNK, HID), jnp.float32),
        pltpu.VMEM((ECHUNK, HID), jnp.float32),
        pltpu.VMEM((ECHUNK, HID), jnp.float32),
        pltpu.VMEM((ECHUNK,), jnp.int32),
        pltpu.VMEM((ECHUNK, HID), jnp.float32),
        pltpu.VMEM((ECHUNK, HID), jnp.float32),
        pltpu.VMEM((ECHUNK, HID), jnp.float32),
        pltpu.VMEM((ECHUNK,), jnp.int32),
    ]
    sems = [pltpu.SemaphoreType.DMA] * 6
    if with_deg:
        outs.append(_f32(NC, NPAD, 16))
        scratch += [pltpu.VMEM((ECHUNK, 16), jnp.float32),
                    pltpu.VMEM((ECHUNK, 16), jnp.float32)]
        sems += [pltpu.SemaphoreType.DMA] * 2
    scratch.append(pltpu.VMEM_SHARED((NPAD, HID), jnp.float32))
    if with_deg:
        scratch.append(pltpu.VMEM_SHARED((NPAD, 16), jnp.float32))
    return pl.kernel(
        functools.partial(_sc_edge_body, with_deg),
        out_type=tuple(outs),
        mesh=_mesh(),
        compiler_params=pltpu.CompilerParams(needs_layout_passes=False),
        scratch_types=scratch + sems,
    )


_sc_edge = _make_sc_edge(False)

DCH = 80
DCHUNKS = EPW // DCH


def _sc_deg_body(dst, deg16, ixall, ix0, ix1, ones, deg_sh, sd0, sd1):
    cid = lax.axis_index("c")
    sid = lax.axis_index("s")
    wid = cid * NS + sid
    row0 = sid * RPT
    ixs = (ix0, ix1)
    sds = (sd0, sd1)
    pltpu.sync_copy(dst.at[pl.ds(wid * EPW, EPW)], ixall)
    zv = jnp.zeros((16,), jnp.float32)
    for i in range(DCH):
        ones[i, :] = zv
    for k in range(RPT // DCH):
        pltpu.sync_copy(ones, deg_sh.at[pl.ds(row0 + k * DCH, DCH)])
    for i in range(DCH):
        ones[i, :] = jnp.full((16,), 1.0, jnp.float32)
    plsc.subcore_barrier()
    for t in range(DCHUNKS):
        ix = ixs[t % 2]
        if t >= 2:
            pltpu.make_async_copy(ones, deg_sh.at[ix], sds[t % 2]).wait()
        off = t * DCH
        for k in range(DCH // 16):
            ix[pl.ds(16 * k, 16)] = ixall[pl.ds(off + 16 * k, 16)]
        pltpu.async_copy(ones, deg_sh.at[ix], sds[t % 2], add=True)
    pltpu.make_async_copy(ones, deg_sh.at[ix0], sd0).wait()
    pltpu.make_async_copy(ones, deg_sh.at[ix1], sd1).wait()
    plsc.subcore_barrier()
    pltpu.sync_copy(deg_sh.at[pl.ds(row0, RPT)],
                    deg16.at[cid, pl.ds(row0, RPT)])


_sc_deg = pl.kernel(
    _sc_deg_body,
    out_type=(_f32(NC, NPAD, 16),),
    mesh=_mesh(),
    compiler_params=pltpu.CompilerParams(needs_layout_passes=False),
    scratch_types=[
        pltpu.VMEM((EPW,), jnp.int32),
        pltpu.VMEM((DCH,), jnp.int32),
        pltpu.VMEM((DCH,), jnp.int32),
        pltpu.VMEM((DCH, 16), jnp.float32),
        pltpu.VMEM_SHARED((NPAD, 16), jnp.float32),
        pltpu.SemaphoreType.DMA,
        pltpu.SemaphoreType.DMA,
    ],
)


def _sc_query_body(s_idx, o_idx, q1, q2, auxf, x_out, g_out,
                   sv, ov, xb0, gb0, xb1, gb1, aux_v,
                   sq0, swx0, swg0, sq1b, swx1, swg1):
    cid = lax.axis_index("c")
    sid = lax.axis_index("s")
    wid = cid * NS + sid
    qbase = wid * QPW
    sets = ((xb0, gb0, sq0, swx0, swg0),
            (xb1, gb1, sq1b, swx1, swg1))

    pltpu.sync_copy(s_idx.at[pl.ds(qbase, QPW)], sv)
    pltpu.sync_copy(o_idx.at[pl.ds(qbase, QPW)], ov)
    pltpu.sync_copy(auxf, aux_v)
    iota8 = lax.iota(jnp.int32, 16) * 8

    def start_q1(S, t):
        pltpu.async_copy(q1.at[sv.at[pl.ds(t * QCHUNK, QCHUNK)]], S[0], S[2])

    def wait_q(S):
        pltpu.make_async_copy(q1.at[sv.at[pl.ds(0, QCHUNK)]],
                              S[0], S[2]).wait()

    def start_q2(S, t):
        pltpu.async_copy(q2.at[ov.at[pl.ds(t * QCHUNK, QCHUNK)]], S[0], S[2],
                         add=True)

    def wait_wx(S):
        pltpu.make_async_copy(S[0], x_out.at[pl.ds(0, QCHUNK)], S[3]).wait()

    def wait_wg(S):
        pltpu.make_async_copy(S[1], g_out.at[pl.ds(0, QCHUNK * 8)],
                              S[4]).wait()

    def feats_and_write(S, t):
        xb, gb = S[0], S[1]
        off = t * QCHUNK
        for g in range(QCHUNK // 16):
            rs = sv[pl.ds(off + 16 * g, 16)]
            ro = ov[pl.ds(off + 16 * g, 16)]
            fs = [plsc.load_gather(aux_v, [rs + (j * N_NODES)])
                  for j in range(NAUX)]
            fo = [plsc.load_gather(aux_v, [ro + (j * N_NODES)])
                  for j in range(NAUX)]
            d0 = fo[0] - fs[0]
            d1 = fo[1] - fs[1]
            d2 = fo[2] - fs[2]
            dist2 = d0 * d0 + d1 * d1 + d2 * d2
            cosv = fs[6] * fo[6] + fs[7] * fo[7] + fs[8] * fo[8]
            feats = [d0, d1, d2, dist2,
                     fo[3] - fs[3], fo[4] - fs[4], fo[5] - fs[5], cosv]
            gidx = iota8 + (g * 16 * 8)
            for f in range(8):
                plsc.store_scatter(gb, [gidx + f], feats[f])
        base = qbase + off
        pltpu.async_copy(xb, x_out.at[pl.ds(base, QCHUNK)], S[3])
        pltpu.async_copy(gb, g_out.at[pl.ds(base * 8, QCHUNK * 8)], S[4])

    start_q1(sets[0], 0)

    def body(i, carry):
        t0 = 2 * i
        A, B = sets

        @pl.when(i > 0)
        def _():
            wait_wx(B)
            wait_wg(B)

        start_q1(B, t0 + 1)
        wait_q(A)
        start_q2(A, t0)
        wait_q(A)

        @pl.when(i > 0)
        def _():
            wait_wg(A)

        feats_and_write(A, t0)
        wait_q(B)
        start_q2(B, t0 + 1)
        wait_q(B)
        feats_and_write(B, t0 + 1)

        @pl.when(i < (QCHUNKS // 2 - 1))
        def _():
            wait_wx(A)
            start_q1(A, t0 + 2)

        return carry

    lax.fori_loop(0, QCHUNKS // 2, body, 0)
    wait_wx(sets[0])
    wait_wg(sets[0])
    wait_wx(sets[1])
    wait_wg(sets[1])


_sc_query = pl.kernel(
    _sc_query_body,
    out_type=(_f32(QPAD, HID), _f32(QPAD * 8)),
    mesh=_mesh(),
    compiler_params=pltpu.CompilerParams(needs_layout_passes=False),
    scratch_types=[
        pltpu.VMEM((QPW,), jnp.int32),
        pltpu.VMEM((QPW,), jnp.int32),
        pltpu.VMEM((QCHUNK, HID), jnp.float32),
        pltpu.VMEM((QCHUNK * 8,), jnp.float32),
        pltpu.VMEM((QCHUNK, HID), jnp.float32),
        pltpu.VMEM((QCHUNK * 8,), jnp.float32),
        pltpu.VMEM((NAUX * N_NODES,), jnp.float32),
        pltpu.SemaphoreType.DMA,
        pltpu.SemaphoreType.DMA,
        pltpu.SemaphoreType.DMA,
        pltpu.SemaphoreType.DMA,
        pltpu.SemaphoreType.DMA,
        pltpu.SemaphoreType.DMA,
    ],
)


RB = 1024
RBU = 1000


def _enc_body(clip, geom, geomT, cW1, cb1, cW2, cb2, fWz, fWg, fb1, fW2, fb2,
              mWa, mWb, mb1, h_out, aux_out, a_out, b_out):
    t = jnp.maximum(jnp.dot(clip[...], cW1[...],
                            preferred_element_type=jnp.float32, precision=lax.Precision.HIGHEST) + cb1[...], 0.0)
    z = jnp.maximum(jnp.dot(t, cW2[...],
                            preferred_element_type=jnp.float32, precision=lax.Precision.HIGHEST) + cb2[...], 0.0)
    u = jnp.maximum(jnp.dot(z, fWz[...], preferred_element_type=jnp.float32, precision=lax.Precision.HIGHEST)
                    + jnp.dot(geom[...], fWg[...],
                              preferred_element_type=jnp.float32, precision=lax.Precision.HIGHEST)
                    + fb1[...], 0.0)
    h = jnp.maximum(jnp.dot(u, fW2[...],
                            preferred_element_type=jnp.float32, precision=lax.Precision.HIGHEST) + fb2[...], 0.0)
    h_out[...] = h
    a_out[...] = jnp.dot(h, mWa[...], preferred_element_type=jnp.float32, precision=lax.Precision.HIGHEST)
    b_out[...] = jnp.dot(h, mWb[...],
                         preferred_element_type=jnp.float32, precision=lax.Precision.HIGHEST) + mb1[...]
    gT = geomT[...]
    c = gT[0:3, :]
    lsz = jnp.log(jnp.maximum(gT[3:6, :], 1e-6))
    n = gT[15:18, :]
    nn = jnp.sqrt(jnp.sum(n * n, axis=0, keepdims=True))
    un = n / jnp.maximum(nn, 1e-8)
    aux_out[...] = jnp.concatenate([c, lsz, un], axis=0)


def _tc_encoder(clip_x, geom_x, geom_T, p):
    full = lambda s: pl.BlockSpec(s, lambda i: (0, 0))
    return pl.pallas_call(
        _enc_body,
        grid=(NPAD // RB,),
        in_specs=[
            pl.BlockSpec((RB, 512), lambda i: (i, 0)),
            pl.BlockSpec((RB, 18), lambda i: (i, 0)),
            pl.BlockSpec((18, RB), lambda i: (0, i)),
            full((512, 512)), full((1, 512)), full((512, 256)), full((1, 256)),
            full((256, HID)), full((18, HID)), full((1, HID)),
            full((HID, HID)), full((1, HID)),
            full((HID, HID)), full((HID, HID)), full((1, HID)),
        ],
        out_specs=[
            pl.BlockSpec((RB, HID), lambda i: (i, 0)),
            pl.BlockSpec((NAUX, RB), lambda i: (0, i)),
            pl.BlockSpec((RB, HID), lambda i: (i, 0)),
            pl.BlockSpec((RB, HID), lambda i: (i, 0)),
        ],
        out_shape=(_f32(N_NODES, HID), _f32(NAUX, N_NODES),
                   _f32(N_NODES, HID), _f32(N_NODES, HID)),
    )(clip_x, geom_x, geom_T,
      p['clip_W1'], p['clip_b1'][None], p['clip_W2'], p['clip_b2'][None],
      p['fuse_W1'][:256], p['fuse_W1'][256:], p['fuse_b1'][None],
      p['fuse_W2'], p['fuse_b2'][None],
      p['layers'][0]['msg_W1'][:HID], p['layers'][0]['msg_W1'][HID:],
      p['layers'][0]['msg_b1'][None])


def _upd_body(parts, degp, h, mW2, mb2, uWa, uWb, ub1, uW2, ub2,
              nWa, nWb, nb, h_out, a_out, b_out):
    aggpre = parts[0] + parts[1]
    deg = degp[0, :, 0:1] + degp[1, :, 0:1]
    agg = jnp.dot(aggpre, mW2[...],
                  preferred_element_type=jnp.float32, precision=lax.Precision.HIGHEST) + deg * mb2[...]
    h0 = h[...]
    u = jnp.maximum(jnp.dot(h0, uWa[...], preferred_element_type=jnp.float32, precision=lax.Precision.HIGHEST)
                    + jnp.dot(agg, uWb[...], preferred_element_type=jnp.float32, precision=lax.Precision.HIGHEST)
                    + ub1[...], 0.0)
    hn = h0 + jnp.dot(u, uW2[...],
                      preferred_element_type=jnp.float32, precision=lax.Precision.HIGHEST) + ub2[...]
    h_out[...] = hn
    a_out[...] = jnp.dot(hn, nWa[...], preferred_element_type=jnp.float32, precision=lax.Precision.HIGHEST)
    b_out[...] = jnp.dot(hn, nWb[...],
                         preferred_element_type=jnp.float32, precision=lax.Precision.HIGHEST) + nb[...]


def _tc_update(parts, degp, h, lp, nWa, nWb, nb):
    full = lambda s: pl.BlockSpec(s, lambda i: (0, 0))
    return pl.pallas_call(
        _upd_body,
        grid=(N_NODES // RBU,),
        in_specs=[
            pl.BlockSpec((NC, RBU, HID), lambda i: (0, i, 0)),
            pl.BlockSpec((NC, RBU, 16), lambda i: (0, i, 0)),
            pl.BlockSpec((RBU, HID), lambda i: (i, 0)),
            full((HID, HID)), full((1, HID)),
            full((HID, HID)), full((HID, HID)), full((1, HID)),
            full((HID, HID)), full((1, HID)),
            full((HID, HID)), full((HID, HID)), full((1, HID)),
        ],
        out_specs=[
            pl.BlockSpec((RBU, HID), lambda i: (i, 0)),
            pl.BlockSpec((RBU, HID), lambda i: (i, 0)),
            pl.BlockSpec((RBU, HID), lambda i: (i, 0)),
        ],
        out_shape=(_f32(N_NODES, HID), _f32(N_NODES, HID), _f32(N_NODES, HID)),
    )(parts, degp, h,
      lp['msg_W2'], lp['msg_b2'][None],
      lp['upd_W1'][:HID], lp['upd_W1'][HID:], lp['upd_b1'][None],
      lp['upd_W2'], lp['upd_b2'][None],
      nWa, nWb, nb)


QB = 2000


def _head_body(x, g8, W1c, W2, b2, out):
    gv = g8[...]
    dist = jnp.sqrt(gv[:, 3:4] + 1e-12)
    log_dist = jnp.log1p(dist)
    rel = jnp.concatenate([gv[:, 0:3], log_dist, gv[:, 4:8]], axis=-1)
    z = jnp.maximum(x[...] + jnp.dot(rel, W1c[...],
                                     preferred_element_type=jnp.float32, precision=lax.Precision.HIGHEST), 0.0)
    out[...] = jnp.dot(z, W2[...],
                       preferred_element_type=jnp.float32, precision=lax.Precision.HIGHEST) + b2[...]


def _tc_head(X, G8, p):
    full = lambda s: pl.BlockSpec(s, lambda i: (0, 0))
    return pl.pallas_call(
        _head_body,
        grid=(N_QUERIES // QB,),
        in_specs=[
            pl.BlockSpec((QB, HID), lambda i: (i, 0)),
            pl.BlockSpec((QB, 8), lambda i: (i, 0)),
            full((8, HID)), full((HID, REL)), full((1, REL)),
        ],
        out_specs=pl.BlockSpec((QB, REL), lambda i: (i, 0)),
        out_shape=_f32(N_QUERIES, REL),
    )(X, G8, p['head_W1'][2 * HID:], p['head_W2'], p['head_b2'][None])


def kernel(clip_x, geom_x, params, graph_edges, query_pairs):
    src = graph_edges[:, 0].astype(jnp.int32)
    dst = graph_edges[:, 1].astype(jnp.int32)
    qp = jnp.pad(query_pairs, ((0, QPAD - N_QUERIES), (0, 0)))
    s_idx = qp[:, 0].astype(jnp.int32)
    o_idx = qp[:, 1].astype(jnp.int32)
    geom_T = geom_x.T

    h, auxT, A, B = _tc_encoder(clip_x, geom_x, geom_T, params)
    auxf = auxT.reshape(NAUX * N_NODES)

    (degp,) = _sc_deg(dst)
    for l in range(3):
        lp = params['layers'][l]
        (parts,) = _sc_edge(src, dst, A, B)
        if l < 2:
            nxt = params['layers'][l + 1]
            nWa = nxt['msg_W1'][:HID]
            nWb = nxt['msg_W1'][HID:]
            nb = nxt['msg_b1'][None]
        else:
            nWa = params['head_W1'][:HID]
            nWb = params['head_W1'][HID:2 * HID]
            nb = params['head_b1'][None]
        h, A, B = _tc_update(parts, degp, h, lp, nWa, nWb, nb)

    X, Gflat = _sc_query(s_idx, o_idx, A, B, auxf)
    return _tc_head(X, Gflat.reshape(QPAD, 8), params)


# speedup vs baseline: 1.3506x; 1.3506x over previous
import functools

import jax
import jax.numpy as jnp
from jax import lax
from jax.experimental import pallas as pl
from jax.experimental.pallas import tpu as pltpu
from jax.experimental.pallas import tpu_sc as plsc

N_NODES = 10000
N_EDGES = 320000
N_QUERIES = 200000
HID = 128
REL = 51

NC = 2
NS = 16
NW = NC * NS

NPAD = 10240
RPT = NPAD // NS

EPW = N_EDGES // NW
ECHUNK = 40
ECHUNKS = EPW // ECHUNK

QPAD = 200704
QPW = QPAD // NW
QCHUNK = 64
QCHUNKS = QPW // QCHUNK
NAUX = 9

_mesh = functools.partial(
    plsc.VectorSubcoreMesh, core_axis_name="c", subcore_axis_name="s",
    num_cores=NC, num_subcores=NS)


def _f32(*shape):
    return jax.ShapeDtypeStruct(shape, jnp.float32)


EBLK = 2000
EBCH = EBLK // ECHUNK


def _sc_edge_body(with_deg, *refs):
    if with_deg:
        (src, dst, a_tab, b_tab, parts, deg16,
         isb, idb, ar0, br0, pb0, ids0, ar1, br1, pb1, ids1, ones, zc,
         acc_sh, deg_sh,
         sga0, sgb0, ssc0, sga1, sgb1, ssc1, sd0, sd1) = refs
    else:
        (src, dst, a_tab, b_tab, parts,
         isb, idb, ar0, br0, pb0, ids0, ar1, br1, pb1, ids1,
         acc_sh,
         sga0, sgb0, ssc0, sga1, sgb1, ssc1) = refs
        sd0 = sd1 = None
    cid = lax.axis_index("c")
    sid = lax.axis_index("s")
    wid = cid * NS + sid
    row0 = sid * RPT
    sets = ((ar0, br0, pb0, ids0, sga0, sgb0, ssc0, sd0),
            (ar1, br1, pb1, ids1, sga1, sgb1, ssc1, sd1))

    zv = jnp.zeros((16,), jnp.float32)
    zi = jnp.zeros((16,), jnp.int32)
    for i in range(ECHUNK):
        for j in range(HID // 16):
            pb0[i, pl.ds(j * 16, 16)] = zv
            pb1[i, pl.ds(j * 16, 16)] = zv
    for j in range(3):
        off = (0, 16, 24)[j]
        ids0[pl.ds(off, 16)] = zi
        ids1[pl.ds(off, 16)] = zi
    for k in range(RPT // ECHUNK):
        pltpu.sync_copy(pb0, acc_sh.at[pl.ds(row0 + k * ECHUNK, ECHUNK)])
    pltpu.async_copy(pb0, acc_sh.at[ids0], ssc0, add=True)
    pltpu.async_copy(pb1, acc_sh.at[ids1], ssc1, add=True)
    if with_deg:
        for i in range(ECHUNK):
            zc[i, :] = zv
        for k in range(RPT // ECHUNK):
            pltpu.sync_copy(zc, deg_sh.at[pl.ds(row0 + k * ECHUNK, ECHUNK)])
        pltpu.async_copy(zc, deg_sh.at[ids0], sd0, add=True)
        pltpu.async_copy(zc, deg_sh.at[ids1], sd1, add=True)
        for i in range(ECHUNK):
            ones[i, :] = jnp.full((16,), 1.0, jnp.float32)
    plsc.subcore_barrier()

    def start_g(S, t):
        off = t * ECHUNK
        pltpu.async_copy(a_tab.at[isb.at[pl.ds(off, ECHUNK)]], S[0], S[4])
        pltpu.async_copy(b_tab.at[idb.at[pl.ds(off, ECHUNK)]], S[1], S[5])

    def wait_g(S):
        pltpu.make_async_copy(a_tab.at[isb.at[pl.ds(0, ECHUNK)]],
                              S[0], S[4]).wait()
        pltpu.make_async_copy(b_tab.at[idb.at[pl.ds(0, ECHUNK)]],
                              S[1], S[5]).wait()

    def wait_s(S):
        pltpu.make_async_copy(S[2], acc_sh.at[S[3]], S[6]).wait()
        if with_deg:
            pltpu.make_async_copy(ones, deg_sh.at[S[3]], S[7]).wait()

    def finish(S, t):
        ar, br, pb, ids = S[0], S[1], S[2], S[3]
        off = t * ECHUNK
        ids[pl.ds(0, 16)] = idb[pl.ds(off, 16)]
        ids[pl.ds(16, 16)] = idb[pl.ds(off + 16, 16)]
        ids[pl.ds(24, 16)] = idb[pl.ds(off + 24, 16)]

        def row(i, c):
            for j in range(HID // 16):
                sl = pl.ds(j * 16, 16)
                pb[i, sl] = jnp.maximum(ar[i, sl] + br[i, sl], 0.0)
            return c

        lax.fori_loop(0, ECHUNK, row, 0)
        pltpu.async_copy(pb, acc_sh.at[ids], S[6], add=True)
        if with_deg:
            pltpu.async_copy(ones, deg_sh.at[ids], S[7], add=True)

    def stage(st, carry):
        base = wid * EPW + st * EBLK
        pltpu.sync_copy(src.at[pl.ds(base, EBLK)], isb)
        pltpu.sync_copy(dst.at[pl.ds(base, EBLK)], idb)
        start_g(sets[0], 0)

        def body(i, c):
            t0 = 2 * i
            A, B = sets
            start_g(B, t0 + 1)
            wait_g(A)
            wait_s(A)
            finish(A, t0)

            @pl.when(i < (EBCH // 2 - 1))
            def _():
                start_g(A, t0 + 2)

            wait_g(B)
            wait_s(B)
            finish(B, t0 + 1)
            return c

        lax.fori_loop(0, EBCH // 2, body, 0)
        return carry

    lax.fori_loop(0, EPW // EBLK, stage, 0)
    wait_s(sets[0])
    wait_s(sets[1])
    plsc.subcore_barrier()
    pltpu.sync_copy(acc_sh.at[pl.ds(row0, RPT)],
                    parts.at[cid, pl.ds(row0, RPT)])
    if with_deg:
        pltpu.sync_copy(deg_sh.at[pl.ds(row0, RPT)],
                        deg16.at[cid, pl.ds(row0, RPT)])


def _make_sc_edge(with_deg):
    outs = [_f32(NC, NPAD, HID)]
    scratch = [
        pltpu.VMEM((EBLK,), jnp.int32),
        pltpu.VMEM((EBLK,), jnp.int32),
        pltpu.VMEM((ECHUNK, HID), jnp.float32),
        pltpu.VMEM((ECHUNK, HID), jnp.float32),
        pltpu.VMEM((ECHUNK, HID), jnp.float32),
        pltpu.VMEM((ECHUNK,), jnp.int32),
        pltpu.VMEM((ECHUNK, HID), jnp.float32),
        pltpu.VMEM((ECHUNK, HID), jnp.float32),
        pltpu.VMEM((ECHUNK, HID), jnp.float32),
        pltpu.VMEM((ECHUNK,), jnp.int32),
    ]
    sems = [pltpu.SemaphoreType.DMA] * 6
    if with_deg:
        outs.append(_f32(NC, NPAD, 16))
        scratch += [pltpu.VMEM((ECHUNK, 16), jnp.float32),
                    pltpu.VMEM((ECHUNK, 16), jnp.float32)]
        sems += [pltpu.SemaphoreType.DMA] * 2
    scratch.append(pltpu.VMEM_SHARED((NPAD, HID), jnp.float32))
    if with_deg:
        scratch.append(pltpu.VMEM_SHARED((NPAD, 16), jnp.float32))
    return pl.kernel(
        functools.partial(_sc_edge_body, with_deg),
        out_type=tuple(outs),
        mesh=_mesh(),
        compiler_params=pltpu.CompilerParams(needs_layout_passes=False),
        scratch_types=scratch + sems,
    )


_sc_edge = _make_sc_edge(False)

DCH = 80
DCHUNKS = EPW // DCH


def _sc_deg_body(dst, deg16, ixall, ix0, ix1, ones, deg_sh, sd0, sd1):
    cid = lax.axis_index("c")
    sid = lax.axis_index("s")
    wid = cid * NS + sid
    row0 = sid * RPT
    ixs = (ix0, ix1)
    sds = (sd0, sd1)
    pltpu.sync_copy(dst.at[pl.ds(wid * EPW, EPW)], ixall)
    zv = jnp.zeros((16,), jnp.float32)
    for i in range(DCH):
        ones[i, :] = zv
    for k in range(RPT // DCH):
        pltpu.sync_copy(ones, deg_sh.at[pl.ds(row0 + k * DCH, DCH)])
    for i in range(DCH):
        ones[i, :] = jnp.full((16,), 1.0, jnp.float32)
    plsc.subcore_barrier()
    for t in range(DCHUNKS):
        ix = ixs[t % 2]
        if t >= 2:
            pltpu.make_async_copy(ones, deg_sh.at[ix], sds[t % 2]).wait()
        off = t * DCH
        for k in range(DCH // 16):
            ix[pl.ds(16 * k, 16)] = ixall[pl.ds(off + 16 * k, 16)]
        pltpu.async_copy(ones, deg_sh.at[ix], sds[t % 2], add=True)
    pltpu.make_async_copy(ones, deg_sh.at[ix0], sd0).wait()
    pltpu.make_async_copy(ones, deg_sh.at[ix1], sd1).wait()
    plsc.subcore_barrier()
    pltpu.sync_copy(deg_sh.at[pl.ds(row0, RPT)],
                    deg16.at[cid, pl.ds(row0, RPT)])


_sc_deg = pl.kernel(
    _sc_deg_body,
    out_type=(_f32(NC, NPAD, 16),),
    mesh=_mesh(),
    compiler_params=pltpu.CompilerParams(needs_layout_passes=False),
    scratch_types=[
        pltpu.VMEM((EPW,), jnp.int32),
        pltpu.VMEM((DCH,), jnp.int32),
        pltpu.VMEM((DCH,), jnp.int32),
        pltpu.VMEM((DCH, 16), jnp.float32),
        pltpu.VMEM_SHARED((NPAD, 16), jnp.float32),
        pltpu.SemaphoreType.DMA,
        pltpu.SemaphoreType.DMA,
    ],
)


def _sc_query_body(s_idx, o_idx, q1, q2, auxf, x_out, g_out,
                   sv, ov, xb0, gb0, xb1, gb1, aux_v,
                   sq0, swx0, swg0, sq1b, swx1, swg1):
    cid = lax.axis_index("c")
    sid = lax.axis_index("s")
    wid = cid * NS + sid
    qbase = wid * QPW
    sets = ((xb0, gb0, sq0, swx0, swg0),
            (xb1, gb1, sq1b, swx1, swg1))

    pltpu.sync_copy(s_idx.at[pl.ds(qbase, QPW)], sv)
    pltpu.sync_copy(o_idx.at[pl.ds(qbase, QPW)], ov)
    pltpu.sync_copy(auxf, aux_v)
    iota8 = lax.iota(jnp.int32, 16) * 8

    def start_q1(S, t):
        pltpu.async_copy(q1.at[sv.at[pl.ds(t * QCHUNK, QCHUNK)]], S[0], S[2])

    def wait_q(S):
        pltpu.make_async_copy(q1.at[sv.at[pl.ds(0, QCHUNK)]],
                              S[0], S[2]).wait()

    def start_q2(S, t):
        pltpu.async_copy(q2.at[ov.at[pl.ds(t * QCHUNK, QCHUNK)]], S[0], S[2],
                         add=True)

    def wait_wx(S):
        pltpu.make_async_copy(S[0], x_out.at[pl.ds(0, QCHUNK)], S[3]).wait()

    def wait_wg(S):
        pltpu.make_async_copy(S[1], g_out.at[pl.ds(0, QCHUNK * 8)],
                              S[4]).wait()

    def feats_and_write(S, t):
        xb, gb = S[0], S[1]
        off = t * QCHUNK
        for g in range(QCHUNK // 16):
            rs = sv[pl.ds(off + 16 * g, 16)]
            ro = ov[pl.ds(off + 16 * g, 16)]
            fs = [plsc.load_gather(aux_v, [rs + (j * N_NODES)])
                  for j in range(NAUX)]
            fo = [plsc.load_gather(aux_v, [ro + (j * N_NODES)])
                  for j in range(NAUX)]
            d0 = fo[0] - fs[0]
            d1 = fo[1] - fs[1]
            d2 = fo[2] - fs[2]
            dist2 = d0 * d0 + d1 * d1 + d2 * d2
            cosv = fs[6] * fo[6] + fs[7] * fo[7] + fs[8] * fo[8]
            feats = [d0, d1, d2, dist2,
                     fo[3] - fs[3], fo[4] - fs[4], fo[5] - fs[5], cosv]
            gidx = iota8 + (g * 16 * 8)
            for f in range(8):
                plsc.store_scatter(gb, [gidx + f], feats[f])
        base = qbase + off
        pltpu.async_copy(xb, x_out.at[pl.ds(base, QCHUNK)], S[3])
        pltpu.async_copy(gb, g_out.at[pl.ds(base * 8, QCHUNK * 8)], S[4])

    start_q1(sets[0], 0)

    def body(i, carry):
        t0 = 2 * i
        A, B = sets

        @pl.when(i > 0)
        def _():
            wait_wx(B)
            wait_wg(B)

        start_q1(B, t0 + 1)
        wait_q(A)
        start_q2(A, t0)
        wait_q(A)

        @pl.when(i > 0)
        def _():
            wait_wg(A)

        feats_and_write(A, t0)
        wait_q(B)
        start_q2(B, t0 + 1)
        wait_q(B)
        feats_and_write(B, t0 + 1)

        @pl.when(i < (QCHUNKS // 2 - 1))
        def _():
            wait_wx(A)
            start_q1(A, t0 + 2)

        return carry

    lax.fori_loop(0, QCHUNKS // 2, body, 0)
    wait_wx(sets[0])
    wait_wg(sets[0])
    wait_wx(sets[1])
    wait_wg(sets[1])


_sc_query = pl.kernel(
    _sc_query_body,
    out_type=(_f32(QPAD, HID), _f32(QPAD * 8)),
    mesh=_mesh(),
    compiler_params=pltpu.CompilerParams(needs_layout_passes=False),
    scratch_types=[
        pltpu.VMEM((QPW,), jnp.int32),
        pltpu.VMEM((QPW,), jnp.int32),
        pltpu.VMEM((QCHUNK, HID), jnp.float32),
        pltpu.VMEM((QCHUNK * 8,), jnp.float32),
        pltpu.VMEM((QCHUNK, HID), jnp.float32),
        pltpu.VMEM((QCHUNK * 8,), jnp.float32),
        pltpu.VMEM((NAUX * N_NODES,), jnp.float32),
        pltpu.SemaphoreType.DMA,
        pltpu.SemaphoreType.DMA,
        pltpu.SemaphoreType.DMA,
        pltpu.SemaphoreType.DMA,
        pltpu.SemaphoreType.DMA,
        pltpu.SemaphoreType.DMA,
    ],
)


RB = 1024
RBU = 1000


def _enc_body(clip, geom, geomT, cW1, cb1, cW2, cb2, fWz, fWg, fb1, fW2, fb2,
              mWa, mWb, mb1, h_out, aux_out, a_out, b_out):
    t = jnp.maximum(jnp.dot(clip[...], cW1[...],
                            preferred_element_type=jnp.float32) + cb1[...], 0.0)
    z = jnp.maximum(jnp.dot(t, cW2[...],
                            preferred_element_type=jnp.float32) + cb2[...], 0.0)
    u = jnp.maximum(jnp.dot(z, fWz[...], preferred_element_type=jnp.float32)
                    + jnp.dot(geom[...], fWg[...],
                              preferred_element_type=jnp.float32)
                    + fb1[...], 0.0)
    h = jnp.maximum(jnp.dot(u, fW2[...],
                            preferred_element_type=jnp.float32) + fb2[...], 0.0)
    h_out[...] = h
    a_out[...] = jnp.dot(h, mWa[...], preferred_element_type=jnp.float32)
    b_out[...] = jnp.dot(h, mWb[...],
                         preferred_element_type=jnp.float32) + mb1[...]
    gT = geomT[...]
    c = gT[0:3, :]
    lsz = jnp.log(jnp.maximum(gT[3:6, :], 1e-6))
    n = gT[15:18, :]
    nn = jnp.sqrt(jnp.sum(n * n, axis=0, keepdims=True))
    un = n / jnp.maximum(nn, 1e-8)
    aux_out[...] = jnp.concatenate([c, lsz, un], axis=0)


def _tc_encoder(clip_x, geom_x, geom_T, p):
    full = lambda s: pl.BlockSpec(s, lambda i: (0, 0))
    return pl.pallas_call(
        _enc_body,
        grid=(NPAD // RB,),
        in_specs=[
            pl.BlockSpec((RB, 512), lambda i: (i, 0)),
            pl.BlockSpec((RB, 18), lambda i: (i, 0)),
            pl.BlockSpec((18, RB), lambda i: (0, i)),
            full((512, 512)), full((1, 512)), full((512, 256)), full((1, 256)),
            full((256, HID)), full((18, HID)), full((1, HID)),
            full((HID, HID)), full((1, HID)),
            full((HID, HID)), full((HID, HID)), full((1, HID)),
        ],
        out_specs=[
            pl.BlockSpec((RB, HID), lambda i: (i, 0)),
            pl.BlockSpec((NAUX, RB), lambda i: (0, i)),
            pl.BlockSpec((RB, HID), lambda i: (i, 0)),
            pl.BlockSpec((RB, HID), lambda i: (i, 0)),
        ],
        out_shape=(_f32(N_NODES, HID), _f32(NAUX, N_NODES),
                   _f32(N_NODES, HID), _f32(N_NODES, HID)),
    )(clip_x, geom_x, geom_T,
      p['clip_W1'], p['clip_b1'][None], p['clip_W2'], p['clip_b2'][None],
      p['fuse_W1'][:256], p['fuse_W1'][256:], p['fuse_b1'][None],
      p['fuse_W2'], p['fuse_b2'][None],
      p['layers'][0]['msg_W1'][:HID], p['layers'][0]['msg_W1'][HID:],
      p['layers'][0]['msg_b1'][None])


def _upd_body(parts, degp, h, mW2, mb2, uWa, uWb, ub1, uW2, ub2,
              nWa, nWb, nb, h_out, a_out, b_out):
    aggpre = parts[0] + parts[1]
    deg = degp[0, :, 0:1] + degp[1, :, 0:1]
    agg = jnp.dot(aggpre, mW2[...],
                  preferred_element_type=jnp.float32) + deg * mb2[...]
    h0 = h[...]
    u = jnp.maximum(jnp.dot(h0, uWa[...], preferred_element_type=jnp.float32)
                    + jnp.dot(agg, uWb[...], preferred_element_type=jnp.float32)
                    + ub1[...], 0.0)
    hn = h0 + jnp.dot(u, uW2[...],
                      preferred_element_type=jnp.float32) + ub2[...]
    h_out[...] = hn
    a_out[...] = jnp.dot(hn, nWa[...], preferred_element_type=jnp.float32)
    b_out[...] = jnp.dot(hn, nWb[...],
                         preferred_element_type=jnp.float32) + nb[...]


def _tc_update(parts, degp, h, lp, nWa, nWb, nb):
    full = lambda s: pl.BlockSpec(s, lambda i: (0, 0))
    return pl.pallas_call(
        _upd_body,
        grid=(N_NODES // RBU,),
        in_specs=[
            pl.BlockSpec((NC, RBU, HID), lambda i: (0, i, 0)),
            pl.BlockSpec((NC, RBU, 16), lambda i: (0, i, 0)),
            pl.BlockSpec((RBU, HID), lambda i: (i, 0)),
            full((HID, HID)), full((1, HID)),
            full((HID, HID)), full((HID, HID)), full((1, HID)),
            full((HID, HID)), full((1, HID)),
            full((HID, HID)), full((HID, HID)), full((1, HID)),
        ],
        out_specs=[
            pl.BlockSpec((RBU, HID), lambda i: (i, 0)),
            pl.BlockSpec((RBU, HID), lambda i: (i, 0)),
            pl.BlockSpec((RBU, HID), lambda i: (i, 0)),
        ],
        out_shape=(_f32(N_NODES, HID), _f32(N_NODES, HID), _f32(N_NODES, HID)),
    )(parts, degp, h,
      lp['msg_W2'], lp['msg_b2'][None],
      lp['upd_W1'][:HID], lp['upd_W1'][HID:], lp['upd_b1'][None],
      lp['upd_W2'], lp['upd_b2'][None],
      nWa, nWb, nb)


QB = 2000


def _head_body(x, g8, W1c, W2, b2, out):
    gv = g8[...]
    dist = jnp.sqrt(gv[:, 3:4] + 1e-12)
    log_dist = jnp.log1p(dist)
    rel = jnp.concatenate([gv[:, 0:3], log_dist, gv[:, 4:8]], axis=-1)
    z = jnp.maximum(x[...] + jnp.dot(rel, W1c[...],
                                     preferred_element_type=jnp.float32), 0.0)
    out[...] = jnp.dot(z, W2[...],
                       preferred_element_type=jnp.float32) + b2[...]


def _tc_head(X, G8, p):
    full = lambda s: pl.BlockSpec(s, lambda i: (0, 0))
    return pl.pallas_call(
        _head_body,
        grid=(N_QUERIES // QB,),
        in_specs=[
            pl.BlockSpec((QB, HID), lambda i: (i, 0)),
            pl.BlockSpec((QB, 8), lambda i: (i, 0)),
            full((8, HID)), full((HID, REL)), full((1, REL)),
        ],
        out_specs=pl.BlockSpec((QB, REL), lambda i: (i, 0)),
        out_shape=_f32(N_QUERIES, REL),
    )(X, G8, p['head_W1'][2 * HID:], p['head_W2'], p['head_b2'][None])


def kernel(clip_x, geom_x, params, graph_edges, query_pairs):
    src = graph_edges[:, 0].astype(jnp.int32)
    dst = graph_edges[:, 1].astype(jnp.int32)
    qp = jnp.pad(query_pairs, ((0, QPAD - N_QUERIES), (0, 0)))
    s_idx = qp[:, 0].astype(jnp.int32)
    o_idx = qp[:, 1].astype(jnp.int32)
    geom_T = geom_x.T

    h, auxT, A, B = _tc_encoder(clip_x, geom_x, geom_T, params)
    auxf = auxT.reshape(NAUX * N_NODES)

    (degp,) = _sc_deg(dst)
    for l in range(3):
        lp = params['layers'][l]
        (parts,) = _sc_edge(src, dst, A, B)
        if l < 2:
            nxt = params['layers'][l + 1]
            nWa = nxt['msg_W1'][:HID]
            nWb = nxt['msg_W1'][HID:]
            nb = nxt['msg_b1'][None]
        else:
            nWa = params['head_W1'][:HID]
            nWb = params['head_W1'][HID:2 * HID]
            nb = params['head_b1'][None]
        h, A, B = _tc_update(parts, degp, h, lp, nWa, nWb, nb)

    X, Gflat = _sc_query(s_idx, o_idx, A, B, auxf)
    return _tc_head(X, Gflat.reshape(QPAD, 8), params)


# speedup vs baseline: 1.4043x; 1.0398x over previous
import functools

import jax
import jax.numpy as jnp
from jax import lax
from jax.experimental import pallas as pl
from jax.experimental.pallas import tpu as pltpu
from jax.experimental.pallas import tpu_sc as plsc

N_NODES = 10000
N_EDGES = 320000
N_QUERIES = 200000
HID = 128
REL = 51

NC = 2
NS = 16
NW = NC * NS

NPAD = 10240
RPT = NPAD // NS

EPW = N_EDGES // NW
ECHUNK = 40
ECHUNKS = EPW // ECHUNK

QPAD = 200704
QPW = QPAD // NW
QCHUNK = 64
QCHUNKS = QPW // QCHUNK
NAUX = 9

_mesh = functools.partial(
    plsc.VectorSubcoreMesh, core_axis_name="c", subcore_axis_name="s",
    num_cores=NC, num_subcores=NS)


def _f32(*shape):
    return jax.ShapeDtypeStruct(shape, jnp.float32)


EBLK = 2000
EBCH = EBLK // ECHUNK


def _sc_edge_body(with_deg, *refs):
    if with_deg:
        (src, dst, a_tab, b_tab, parts, deg16,
         isb, idb, ar0, br0, pb0, ids0, ar1, br1, pb1, ids1, ones, zc,
         acc_sh, deg_sh,
         sga0, sgb0, ssc0, sga1, sgb1, ssc1, sd0, sd1) = refs
    else:
        (src, dst, a_tab, b_tab, parts,
         isb, idb, ar0, br0, pb0, ids0, ar1, br1, pb1, ids1,
         acc_sh,
         sga0, sgb0, ssc0, sga1, sgb1, ssc1) = refs
        sd0 = sd1 = None
    cid = lax.axis_index("c")
    sid = lax.axis_index("s")
    wid = cid * NS + sid
    row0 = sid * RPT
    sets = ((ar0, br0, pb0, ids0, sga0, sgb0, ssc0, sd0),
            (ar1, br1, pb1, ids1, sga1, sgb1, ssc1, sd1))

    zv = jnp.zeros((16,), jnp.float32)
    zi = jnp.zeros((16,), jnp.int32)
    for i in range(ECHUNK):
        for j in range(HID // 16):
            pb0[i, pl.ds(j * 16, 16)] = zv
            pb1[i, pl.ds(j * 16, 16)] = zv
    for j in range(3):
        off = (0, 16, 24)[j]
        ids0[pl.ds(off, 16)] = zi
        ids1[pl.ds(off, 16)] = zi
    for k in range(RPT // ECHUNK):
        pltpu.sync_copy(pb0, acc_sh.at[pl.ds(row0 + k * ECHUNK, ECHUNK)])
    pltpu.async_copy(pb0, acc_sh.at[ids0], ssc0, add=True)
    pltpu.async_copy(pb1, acc_sh.at[ids1], ssc1, add=True)
    if with_deg:
        for i in range(ECHUNK):
            zc[i, :] = zv
        for k in range(RPT // ECHUNK):
            pltpu.sync_copy(zc, deg_sh.at[pl.ds(row0 + k * ECHUNK, ECHUNK)])
        pltpu.async_copy(zc, deg_sh.at[ids0], sd0, add=True)
        pltpu.async_copy(zc, deg_sh.at[ids1], sd1, add=True)
        for i in range(ECHUNK):
            ones[i, :] = jnp.full((16,), 1.0, jnp.float32)
    plsc.subcore_barrier()

    def start_g(S, t):
        off = t * ECHUNK
        pltpu.async_copy(a_tab.at[isb.at[pl.ds(off, ECHUNK)]], S[0], S[4])
        pltpu.async_copy(b_tab.at[idb.at[pl.ds(off, ECHUNK)]], S[1], S[5])

    def wait_g(S):
        pltpu.make_async_copy(a_tab.at[isb.at[pl.ds(0, ECHUNK)]],
                              S[0], S[4]).wait()
        pltpu.make_async_copy(b_tab.at[idb.at[pl.ds(0, ECHUNK)]],
                              S[1], S[5]).wait()

    def wait_s(S):
        pltpu.make_async_copy(S[2], acc_sh.at[S[3]], S[6]).wait()
        if with_deg:
            pltpu.make_async_copy(ones, deg_sh.at[S[3]], S[7]).wait()

    def finish(S, t):
        ar, br, pb, ids = S[0], S[1], S[2], S[3]
        off = t * ECHUNK
        ids[pl.ds(0, 16)] = idb[pl.ds(off, 16)]
        ids[pl.ds(16, 16)] = idb[pl.ds(off + 16, 16)]
        ids[pl.ds(24, 16)] = idb[pl.ds(off + 24, 16)]

        def row(i, c):
            for j in range(HID // 16):
                sl = pl.ds(j * 16, 16)
                pb[i, sl] = jnp.maximum(ar[i, sl] + br[i, sl], 0.0)
            return c

        lax.fori_loop(0, ECHUNK, row, 0)
        pltpu.async_copy(pb, acc_sh.at[ids], S[6], add=True)
        if with_deg:
            pltpu.async_copy(ones, deg_sh.at[ids], S[7], add=True)

    def stage(st, carry):
        base = wid * EPW + st * EBLK
        pltpu.sync_copy(src.at[pl.ds(base, EBLK)], isb)
        pltpu.sync_copy(dst.at[pl.ds(base, EBLK)], idb)
        start_g(sets[0], 0)

        def body(i, c):
            t0 = 2 * i
            A, B = sets
            start_g(B, t0 + 1)
            wait_g(A)
            wait_s(A)
            finish(A, t0)

            @pl.when(i < (EBCH // 2 - 1))
            def _():
                start_g(A, t0 + 2)

            wait_g(B)
            wait_s(B)
            finish(B, t0 + 1)
            return c

        lax.fori_loop(0, EBCH // 2, body, 0)
        return carry

    lax.fori_loop(0, EPW // EBLK, stage, 0)
    wait_s(sets[0])
    wait_s(sets[1])
    plsc.subcore_barrier()
    pltpu.sync_copy(acc_sh.at[pl.ds(row0, RPT)],
                    parts.at[cid, pl.ds(row0, RPT)])
    if with_deg:
        pltpu.sync_copy(deg_sh.at[pl.ds(row0, RPT)],
                        deg16.at[cid, pl.ds(row0, RPT)])


def _make_sc_edge(with_deg):
    outs = [_f32(NC, NPAD, HID)]
    scratch = [
        pltpu.VMEM((EBLK,), jnp.int32),
        pltpu.VMEM((EBLK,), jnp.int32),
        pltpu.VMEM((ECHUNK, HID), jnp.float32),
        pltpu.VMEM((ECHUNK, HID), jnp.float32),
        pltpu.VMEM((ECHUNK, HID), jnp.float32),
        pltpu.VMEM((ECHUNK,), jnp.int32),
        pltpu.VMEM((ECHUNK, HID), jnp.float32),
        pltpu.VMEM((ECHUNK, HID), jnp.float32),
        pltpu.VMEM((ECHUNK, HID), jnp.float32),
        pltpu.VMEM((ECHUNK,), jnp.int32),
    ]
    sems = [pltpu.SemaphoreType.DMA] * 6
    if with_deg:
        outs.append(_f32(NC, NPAD, 16))
        scratch += [pltpu.VMEM((ECHUNK, 16), jnp.float32),
                    pltpu.VMEM((ECHUNK, 16), jnp.float32)]
        sems += [pltpu.SemaphoreType.DMA] * 2
    scratch.append(pltpu.VMEM_SHARED((NPAD, HID), jnp.float32))
    if with_deg:
        scratch.append(pltpu.VMEM_SHARED((NPAD, 16), jnp.float32))
    return pl.kernel(
        functools.partial(_sc_edge_body, with_deg),
        out_type=tuple(outs),
        mesh=_mesh(),
        compiler_params=pltpu.CompilerParams(needs_layout_passes=False),
        scratch_types=scratch + sems,
    )


_sc_edge = _make_sc_edge(False)

DCH = 80
DCHUNKS = EPW // DCH


def _sc_deg_body(dst, deg16, ixall, ix0, ix1, ones, deg_sh, sd0, sd1):
    cid = lax.axis_index("c")
    sid = lax.axis_index("s")
    wid = cid * NS + sid
    row0 = sid * RPT
    ixs = (ix0, ix1)
    sds = (sd0, sd1)
    pltpu.sync_copy(dst.at[pl.ds(wid * EPW, EPW)], ixall)
    zv = jnp.zeros((16,), jnp.float32)
    for i in range(DCH):
        ones[i, :] = zv
    for k in range(RPT // DCH):
        pltpu.sync_copy(ones, deg_sh.at[pl.ds(row0 + k * DCH, DCH)])
    for i in range(DCH):
        ones[i, :] = jnp.full((16,), 1.0, jnp.float32)
    plsc.subcore_barrier()
    for t in range(DCHUNKS):
        ix = ixs[t % 2]
        if t >= 2:
            pltpu.make_async_copy(ones, deg_sh.at[ix], sds[t % 2]).wait()
        off = t * DCH
        for k in range(DCH // 16):
            ix[pl.ds(16 * k, 16)] = ixall[pl.ds(off + 16 * k, 16)]
        pltpu.async_copy(ones, deg_sh.at[ix], sds[t % 2], add=True)
    pltpu.make_async_copy(ones, deg_sh.at[ix0], sd0).wait()
    pltpu.make_async_copy(ones, deg_sh.at[ix1], sd1).wait()
    plsc.subcore_barrier()
    pltpu.sync_copy(deg_sh.at[pl.ds(row0, RPT)],
                    deg16.at[cid, pl.ds(row0, RPT)])


_sc_deg = pl.kernel(
    _sc_deg_body,
    out_type=(_f32(NC, NPAD, 16),),
    mesh=_mesh(),
    compiler_params=pltpu.CompilerParams(needs_layout_passes=False),
    scratch_types=[
        pltpu.VMEM((EPW,), jnp.int32),
        pltpu.VMEM((DCH,), jnp.int32),
        pltpu.VMEM((DCH,), jnp.int32),
        pltpu.VMEM((DCH, 16), jnp.float32),
        pltpu.VMEM_SHARED((NPAD, 16), jnp.float32),
        pltpu.SemaphoreType.DMA,
        pltpu.SemaphoreType.DMA,
    ],
)


def _sc_query_body(s_idx, o_idx, q1, q2, auxf, x_out, g_out,
                   sv, ov, xb0, gb0, xb1, gb1, aux_v,
                   sq0, swx0, swg0, sq1b, swx1, swg1):
    cid = lax.axis_index("c")
    sid = lax.axis_index("s")
    wid = cid * NS + sid
    qbase = wid * QPW
    sets = ((xb0, gb0, sq0, swx0, swg0),
            (xb1, gb1, sq1b, swx1, swg1))

    pltpu.sync_copy(s_idx.at[pl.ds(qbase, QPW)], sv)
    pltpu.sync_copy(o_idx.at[pl.ds(qbase, QPW)], ov)
    pltpu.sync_copy(auxf, aux_v)
    iota8 = lax.iota(jnp.int32, 16) * 8

    def start_q1(S, t):
        pltpu.async_copy(q1.at[sv.at[pl.ds(t * QCHUNK, QCHUNK)]], S[0], S[2])

    def wait_q(S):
        pltpu.make_async_copy(q1.at[sv.at[pl.ds(0, QCHUNK)]],
                              S[0], S[2]).wait()

    def start_q2(S, t):
        pltpu.async_copy(q2.at[ov.at[pl.ds(t * QCHUNK, QCHUNK)]], S[0], S[2],
                         add=True)

    def wait_wx(S):
        pltpu.make_async_copy(S[0], x_out.at[pl.ds(0, QCHUNK)], S[3]).wait()

    def wait_wg(S):
        pltpu.make_async_copy(S[1], g_out.at[pl.ds(0, QCHUNK * 8)],
                              S[4]).wait()

    def feats_and_write(S, t):
        xb, gb = S[0], S[1]
        off = t * QCHUNK
        for g in range(QCHUNK // 16):
            rs = sv[pl.ds(off + 16 * g, 16)]
            ro = ov[pl.ds(off + 16 * g, 16)]
            fs = [plsc.load_gather(aux_v, [rs + (j * N_NODES)])
                  for j in range(NAUX)]
            fo = [plsc.load_gather(aux_v, [ro + (j * N_NODES)])
                  for j in range(NAUX)]
            d0 = fo[0] - fs[0]
            d1 = fo[1] - fs[1]
            d2 = fo[2] - fs[2]
            dist2 = d0 * d0 + d1 * d1 + d2 * d2
            cosv = fs[6] * fo[6] + fs[7] * fo[7] + fs[8] * fo[8]
            feats = [d0, d1, d2, dist2,
                     fo[3] - fs[3], fo[4] - fs[4], fo[5] - fs[5], cosv]
            gidx = iota8 + (g * 16 * 8)
            for f in range(8):
                plsc.store_scatter(gb, [gidx + f], feats[f])
        base = qbase + off
        pltpu.async_copy(xb, x_out.at[pl.ds(base, QCHUNK)], S[3])
        pltpu.async_copy(gb, g_out.at[pl.ds(base * 8, QCHUNK * 8)], S[4])

    start_q1(sets[0], 0)

    def body(i, carry):
        t0 = 2 * i
        A, B = sets

        @pl.when(i > 0)
        def _():
            wait_wx(B)
            wait_wg(B)

        start_q1(B, t0 + 1)
        wait_q(A)
        start_q2(A, t0)
        wait_q(A)

        @pl.when(i > 0)
        def _():
            wait_wg(A)

        feats_and_write(A, t0)
        wait_q(B)
        start_q2(B, t0 + 1)
        wait_q(B)
        feats_and_write(B, t0 + 1)

        @pl.when(i < (QCHUNKS // 2 - 1))
        def _():
            wait_wx(A)
            start_q1(A, t0 + 2)

        return carry

    lax.fori_loop(0, QCHUNKS // 2, body, 0)
    wait_wx(sets[0])
    wait_wg(sets[0])
    wait_wx(sets[1])
    wait_wg(sets[1])


_sc_query = pl.kernel(
    _sc_query_body,
    out_type=(_f32(QPAD, HID), _f32(QPAD * 8)),
    mesh=_mesh(),
    compiler_params=pltpu.CompilerParams(needs_layout_passes=False),
    scratch_types=[
        pltpu.VMEM((QPW,), jnp.int32),
        pltpu.VMEM((QPW,), jnp.int32),
        pltpu.VMEM((QCHUNK, HID), jnp.float32),
        pltpu.VMEM((QCHUNK * 8,), jnp.float32),
        pltpu.VMEM((QCHUNK, HID), jnp.float32),
        pltpu.VMEM((QCHUNK * 8,), jnp.float32),
        pltpu.VMEM((NAUX * N_NODES,), jnp.float32),
        pltpu.SemaphoreType.DMA,
        pltpu.SemaphoreType.DMA,
        pltpu.SemaphoreType.DMA,
        pltpu.SemaphoreType.DMA,
        pltpu.SemaphoreType.DMA,
        pltpu.SemaphoreType.DMA,
    ],
)


RB = 2048
RBU = 2000


def _enc_body(clip, geom, geomT, cW1, cb1, cW2, cb2, fWz, fWg, fb1, fW2, fb2,
              mWa, mWb, mb1, h_out, aux_out, a_out, b_out):
    t = jnp.maximum(jnp.dot(clip[...], cW1[...],
                            preferred_element_type=jnp.float32) + cb1[...], 0.0)
    z = jnp.maximum(jnp.dot(t, cW2[...],
                            preferred_element_type=jnp.float32) + cb2[...], 0.0)
    u = jnp.maximum(jnp.dot(z, fWz[...], preferred_element_type=jnp.float32)
                    + jnp.dot(geom[...], fWg[...],
                              preferred_element_type=jnp.float32)
                    + fb1[...], 0.0)
    h = jnp.maximum(jnp.dot(u, fW2[...],
                            preferred_element_type=jnp.float32) + fb2[...], 0.0)
    h_out[...] = h
    a_out[...] = jnp.dot(h, mWa[...], preferred_element_type=jnp.float32)
    b_out[...] = jnp.dot(h, mWb[...],
                         preferred_element_type=jnp.float32) + mb1[...]
    gT = geomT[...]
    c = gT[0:3, :]
    lsz = jnp.log(jnp.maximum(gT[3:6, :], 1e-6))
    n = gT[15:18, :]
    nn = jnp.sqrt(jnp.sum(n * n, axis=0, keepdims=True))
    un = n / jnp.maximum(nn, 1e-8)
    aux_out[...] = jnp.concatenate([c, lsz, un], axis=0)


def _tc_encoder(clip_x, geom_x, geom_T, p):
    full = lambda s: pl.BlockSpec(s, lambda i: (0, 0))
    return pl.pallas_call(
        _enc_body,
        grid=(NPAD // RB,),
        in_specs=[
            pl.BlockSpec((RB, 512), lambda i: (i, 0)),
            pl.BlockSpec((RB, 18), lambda i: (i, 0)),
            pl.BlockSpec((18, RB), lambda i: (0, i)),
            full((512, 512)), full((1, 512)), full((512, 256)), full((1, 256)),
            full((256, HID)), full((18, HID)), full((1, HID)),
            full((HID, HID)), full((1, HID)),
            full((HID, HID)), full((HID, HID)), full((1, HID)),
        ],
        out_specs=[
            pl.BlockSpec((RB, HID), lambda i: (i, 0)),
            pl.BlockSpec((NAUX, RB), lambda i: (0, i)),
            pl.BlockSpec((RB, HID), lambda i: (i, 0)),
            pl.BlockSpec((RB, HID), lambda i: (i, 0)),
        ],
        out_shape=(_f32(N_NODES, HID), _f32(NAUX, N_NODES),
                   _f32(N_NODES, HID), _f32(N_NODES, HID)),
    )(clip_x, geom_x, geom_T,
      p['clip_W1'], p['clip_b1'][None], p['clip_W2'], p['clip_b2'][None],
      p['fuse_W1'][:256], p['fuse_W1'][256:], p['fuse_b1'][None],
      p['fuse_W2'], p['fuse_b2'][None],
      p['layers'][0]['msg_W1'][:HID], p['layers'][0]['msg_W1'][HID:],
      p['layers'][0]['msg_b1'][None])


def _upd_body(parts, degp, h, mW2, mb2, uWa, uWb, ub1, uW2, ub2,
              nWa, nWb, nb, h_out, a_out, b_out):
    aggpre = parts[0] + parts[1]
    deg = degp[0, :, 0:1] + degp[1, :, 0:1]
    agg = jnp.dot(aggpre, mW2[...],
                  preferred_element_type=jnp.float32) + deg * mb2[...]
    h0 = h[...]
    u = jnp.maximum(jnp.dot(h0, uWa[...], preferred_element_type=jnp.float32)
                    + jnp.dot(agg, uWb[...], preferred_element_type=jnp.float32)
                    + ub1[...], 0.0)
    hn = h0 + jnp.dot(u, uW2[...],
                      preferred_element_type=jnp.float32) + ub2[...]
    h_out[...] = hn
    a_out[...] = jnp.dot(hn, nWa[...], preferred_element_type=jnp.float32)
    b_out[...] = jnp.dot(hn, nWb[...],
                         preferred_element_type=jnp.float32) + nb[...]


def _tc_update(parts, degp, h, lp, nWa, nWb, nb):
    full = lambda s: pl.BlockSpec(s, lambda i: (0, 0))
    return pl.pallas_call(
        _upd_body,
        grid=(N_NODES // RBU,),
        in_specs=[
            pl.BlockSpec((NC, RBU, HID), lambda i: (0, i, 0)),
            pl.BlockSpec((NC, RBU, 16), lambda i: (0, i, 0)),
            pl.BlockSpec((RBU, HID), lambda i: (i, 0)),
            full((HID, HID)), full((1, HID)),
            full((HID, HID)), full((HID, HID)), full((1, HID)),
            full((HID, HID)), full((1, HID)),
            full((HID, HID)), full((HID, HID)), full((1, HID)),
        ],
        out_specs=[
            pl.BlockSpec((RBU, HID), lambda i: (i, 0)),
            pl.BlockSpec((RBU, HID), lambda i: (i, 0)),
            pl.BlockSpec((RBU, HID), lambda i: (i, 0)),
        ],
        out_shape=(_f32(N_NODES, HID), _f32(N_NODES, HID), _f32(N_NODES, HID)),
    )(parts, degp, h,
      lp['msg_W2'], lp['msg_b2'][None],
      lp['upd_W1'][:HID], lp['upd_W1'][HID:], lp['upd_b1'][None],
      lp['upd_W2'], lp['upd_b2'][None],
      nWa, nWb, nb)


QB = 5000


def _head_body(x, g8, W1c, W2, b2, out):
    gv = g8[...]
    dist = jnp.sqrt(gv[:, 3:4] + 1e-12)
    log_dist = jnp.log1p(dist)
    rel = jnp.concatenate([gv[:, 0:3], log_dist, gv[:, 4:8]], axis=-1)
    z = jnp.maximum(x[...] + jnp.dot(rel, W1c[...],
                                     preferred_element_type=jnp.float32), 0.0)
    out[...] = jnp.dot(z, W2[...],
                       preferred_element_type=jnp.float32) + b2[...]


def _tc_head(X, G8, p):
    full = lambda s: pl.BlockSpec(s, lambda i: (0, 0))
    return pl.pallas_call(
        _head_body,
        grid=(N_QUERIES // QB,),
        in_specs=[
            pl.BlockSpec((QB, HID), lambda i: (i, 0)),
            pl.BlockSpec((QB, 8), lambda i: (i, 0)),
            full((8, HID)), full((HID, REL)), full((1, REL)),
        ],
        out_specs=pl.BlockSpec((QB, REL), lambda i: (i, 0)),
        out_shape=_f32(N_QUERIES, REL),
    )(X, G8, p['head_W1'][2 * HID:], p['head_W2'], p['head_b2'][None])


def kernel(clip_x, geom_x, params, graph_edges, query_pairs):
    src = graph_edges[:, 0].astype(jnp.int32)
    dst = graph_edges[:, 1].astype(jnp.int32)
    qp = jnp.pad(query_pairs, ((0, QPAD - N_QUERIES), (0, 0)))
    s_idx = qp[:, 0].astype(jnp.int32)
    o_idx = qp[:, 1].astype(jnp.int32)
    geom_T = geom_x.T

    h, auxT, A, B = _tc_encoder(clip_x, geom_x, geom_T, params)
    auxf = auxT.reshape(NAUX * N_NODES)

    (degp,) = _sc_deg(dst)
    for l in range(3):
        lp = params['layers'][l]
        (parts,) = _sc_edge(src, dst, A, B)
        if l < 2:
            nxt = params['layers'][l + 1]
            nWa = nxt['msg_W1'][:HID]
            nWb = nxt['msg_W1'][HID:]
            nb = nxt['msg_b1'][None]
        else:
            nWa = params['head_W1'][:HID]
            nWb = params['head_W1'][HID:2 * HID]
            nb = params['head_b1'][None]
        h, A, B = _tc_update(parts, degp, h, lp, nWa, nWb, nb)

    X, Gflat = _sc_query(s_idx, o_idx, A, B, auxf)
    return _tc_head(X, Gflat.reshape(QPAD, 8), params)
